# baseline (speedup 1.0000x reference)
import functools
import math

import jax
import jax.numpy as jnp
from jax import lax
from jax.experimental import pallas as pl
from jax.experimental.pallas import tpu as pltpu
from jax.experimental.pallas import tpu_sc as plsc

H = 512
I = 128
S = 7
B = 8
L = 2048

LOG10_HALF = math.log(10.0) / 2.0
LOG_TAU = [s * LOG10_HALF for s in range(S)]
TAU = [math.exp(v) for v in LOG_TAU]


def _pick_row(blk, rem):
    rowmask = lax.broadcasted_iota(jnp.int32, blk.shape, 0) == rem
    return jnp.sum(jnp.where(rowmask, blk, 0.0), axis=0, keepdims=True)


def _routing_sc_kernel(bidx_hbm, perm_hbm, meta_hbm, bidx_v, perm_v, meta_v):
    cid = lax.axis_index("c")
    sid = lax.axis_index("s")

    @pl.when(jnp.logical_and(cid == 0, sid == 0))
    def _():
        pltpu.sync_copy(bidx_hbm, bidx_v)
        iota = lax.broadcasted_iota(jnp.int32, (16,), 0)
        zero = jnp.zeros((16,), jnp.int32)
        zs = jnp.int32(0)

        def count_body(k, cnts):
            v = bidx_v[pl.ds(pl.multiple_of(k * 16, 16), 16)]
            return tuple(cnts[b] + jnp.sum((v == b).astype(jnp.int32))
                         for b in range(B))

        cnts = lax.fori_loop(0, L // 16, count_body, (zs,) * B)
        offs = []
        run = zs
        for b in range(B):
            offs.append(run)
            run = run + cnts[b]

        def scatter_body(k, pos):
            base = k * 16
            v = bidx_v[pl.ds(pl.multiple_of(base, 16), 16)]
            ids = iota + base
            newpos = []
            for b in range(B):
                mask = v == b
                mi = mask.astype(jnp.int32)
                ranks = plsc.cumsum(mi)
                plsc.store_scatter(perm_v, [pos[b] + ranks - 1], ids, mask=mask)
                newpos.append(pos[b] + jnp.sum(mi))
            return tuple(newpos)

        lax.fori_loop(0, L // 16, scatter_body, tuple(offs))

        cnt_lane = zero
        off_lane = zero
        for b in range(B):
            sel = iota == b
            cnt_lane = jnp.where(sel, cnts[b], cnt_lane)
            off_lane = jnp.where(sel, offs[b], off_lane)
        nsteps = cnts[0]
        for b in range(1, B):
            nsteps = jnp.maximum(nsteps, cnts[b])
        meta_v[pl.ds(0, 16)] = cnt_lane
        meta_v[pl.ds(16, 16)] = off_lane
        meta_v[pl.ds(32, 16)] = zero + nsteps
        pltpu.sync_copy(perm_v, perm_hbm)
        pltpu.sync_copy(meta_v, meta_hbm)


def _route_events(batch_idx):
    return pl.kernel(
        _routing_sc_kernel,
        out_type=(
            jax.ShapeDtypeStruct((L,), jnp.int32),
            jax.ShapeDtypeStruct((48,), jnp.int32),
        ),
        mesh=plsc.VectorSubcoreMesh(core_axis_name="c", subcore_axis_name="s"),
        scratch_types=[
            pltpu.VMEM((L,), jnp.int32),
            pltpu.VMEM((L,), jnp.int32),
            pltpu.VMEM((48,), jnp.int32),
        ],
        compiler_params=pltpu.CompilerParams(needs_layout_passes=False),
    )(batch_idx)


def _ctgru_batched_kernel(
    perm_ref,
    counts_ref,
    offs_ref,
    nsteps_ref,
    x_ref,
    m_ref,
    w1t_ref,
    b1_ref,
    w2t_ref,
    b2_ref,
    wxc_ref,
    whc_ref,
    bc_ref,
    ws_x_ref,
    ws_h_ref,
    bs_ref,
    loss_ref,
    ratio_ref,
    state_ref,
    lastt_ref,
    acc_ref,
    rz_ref,
):
    state_ref[...] = jnp.zeros((8 * B, H), jnp.float32)
    acc_ref[...] = jnp.zeros((B, I), jnp.float32)
    for b in range(B):
        lastt_ref[b] = 0.0

    def fetch(j):
        xs, ms, acts, ivs = [], [], [], []
        for b in range(B):
            nb = counts_ref[b]
            pos = jnp.maximum(offs_ref[b] + jnp.minimum(j, nb - 1), 0)
            t = perm_ref[pos]
            tbase = (t // 8) * 8
            trem = t - tbase
            xs.append(_pick_row(x_ref[pl.ds(tbase, 8), :], trem))
            ms.append(_pick_row(m_ref[pl.ds(tbase, 8), :], trem))
            active = j < nb
            acts.append(jnp.full((1, 1), active.astype(jnp.float32)))
            ot = t.astype(jnp.float32)
            lt = lastt_ref[b]
            ivs.append(jnp.full((1, 1), ot - lt))
            lastt_ref[b] = jnp.where(active, ot, lt)
        return (jnp.concatenate(xs, axis=0), jnp.concatenate(ms, axis=0),
                jnp.concatenate(acts, axis=0), jnp.concatenate(ivs, axis=0))

    x80, m80, act0, iv0 = fetch(0)
    rz_ref[pl.ds(0, 8), :] = (
        jnp.dot(x80.astype(jnp.bfloat16), wxc_ref[...],
                preferred_element_type=jnp.float32) + bc_ref[...])

    def step(j, carry):
        x8, m8, act, iv = carry
        actb = act > 0.5
        x8b = x8.astype(jnp.bfloat16)

        h8 = state_ref[S * 8:(S + 1) * 8, :]
        hh = [state_ref[s * 8:(s + 1) * 8, :] for s in range(S)]

        parity = pl.multiple_of((j % 2) * 8, 8)
        rz = rz_ref[pl.ds(parity, 8), :]

        new_h = hh[0]
        for s in range(1, S):
            new_h = new_h + hh[s]
        h_next = jnp.where(actb, new_h, h8)

        x8n, m8n, actn, ivn = fetch(j + 1)
        rz_next = (jnp.dot(x8n.astype(jnp.bfloat16), wxc_ref[...],
                           preferred_element_type=jnp.float32)
                   + jnp.dot(h_next.astype(jnp.bfloat16), whc_ref[...],
                             preferred_element_type=jnp.float32)
                   + bc_ref[...])
        other = pl.multiple_of((1 - j % 2) * 8, 8)
        rz_ref[pl.ds(other, 8), :] = rz_next

        h8b = h8.astype(jnp.bfloat16)
        a = jnp.maximum(
            jnp.dot(h8b, w1t_ref[...], preferred_element_type=jnp.float32)
            + b1_ref[...], 0.0)
        p = jnp.dot(a.astype(jnp.bfloat16), w2t_ref[...], preferred_element_type=jnp.float32) + b2_ref[...]
        acc_ref[...] += jnp.abs(x8 - p) * m8 * act

        q = [-jnp.square(rz[:, s * H:(s + 1) * H] - LOG_TAU[s]) for s in range(S)]
        mx = q[0]
        for s in range(1, S):
            mx = jnp.maximum(mx, q[s])
        e = [jnp.exp(q[s] - mx) for s in range(S)]
        den = e[0]
        for s in range(1, S):
            den = den + e[s]
        rsum = e[0] * hh[0]
        for s in range(1, S):
            rsum += e[s] * hh[s]
        rsum = rsum / den

        h_tilde = jnp.tanh(
            jnp.dot(x8b, ws_x_ref[...], preferred_element_type=jnp.float32)
            + jnp.dot(rsum.astype(jnp.bfloat16), ws_h_ref[...], preferred_element_type=jnp.float32)
            + bs_ref[...])

        Z0 = S * H
        qz = [-jnp.square(rz[:, Z0 + s * H:Z0 + (s + 1) * H] - LOG_TAU[s]) for s in range(S)]
        mz = qz[0]
        for s in range(1, S):
            mz = jnp.maximum(mz, qz[s])
        ez = [jnp.exp(qz[s] - mz) for s in range(S)]
        dz = ez[0]
        for s in range(1, S):
            dz = dz + ez[s]

        for s in range(S):
            z_s = ez[s] / dz
            expf = jnp.exp(-iv / TAU[s])
            new_hh_s = ((1.0 - z_s) * hh[s] + z_s * h_tilde) * expf
            state_ref[s * 8:(s + 1) * 8, :] = jnp.where(actb, new_hh_s, hh[s])
        state_ref[S * 8:(S + 1) * 8, :] = h_next
        return (x8n, m8n, actn, ivn)

    lax.fori_loop(0, nsteps_ref[0], step, (x80, m80, act0, iv0))

    loss = jnp.sum(acc_ref[...])
    tot_m = jnp.sum(m_ref[...])
    loss_ref[0] = loss
    ratio_ref[0] = loss / tot_m


def kernel(obs_times, event_pt, sample_idx, X, M, batch_idx, device, T,
           W1, b1, W2, b2, Wr, br, Ws, bs, Wst, bst):
    perm, meta = _route_events(batch_idx)
    counts = meta[0:B]
    offs = meta[16:16 + B]
    nsteps = meta[32:33]

    def split_sh(W):
        Wp = W.reshape(H, S, I + H).transpose(2, 1, 0).reshape(I + H, S * H)
        return Wp[:I], Wp[I:]

    bf = lambda w: w.astype(jnp.bfloat16)
    wr_x, wr_h = split_sh(Wr)
    wt_x, wt_h = split_sh(Wst)
    wxc = jnp.concatenate([wr_x, wt_x], axis=1)
    whc = jnp.concatenate([wr_h, wt_h], axis=1)
    br_p = br.reshape(H, S).T.reshape(1, S * H)
    bt_p = bst.reshape(H, S).T.reshape(1, S * H)
    bc = jnp.concatenate([br_p, bt_p], axis=1)
    out = pl.pallas_call(
        _ctgru_batched_kernel,
        out_shape=(
            jax.ShapeDtypeStruct((1,), jnp.float32),
            jax.ShapeDtypeStruct((1,), jnp.float32),
        ),
        in_specs=[
            pl.BlockSpec(memory_space=pltpu.SMEM) for _ in range(4)
        ] + [
            pl.BlockSpec(memory_space=pltpu.VMEM) for _ in range(12)
        ],
        out_specs=(
            pl.BlockSpec(memory_space=pltpu.SMEM),
            pl.BlockSpec(memory_space=pltpu.SMEM),
        ),
        scratch_shapes=[
            pltpu.VMEM((8 * B, H), jnp.float32),
            pltpu.SMEM((B,), jnp.float32),
            pltpu.VMEM((B, I), jnp.float32),
            pltpu.VMEM((16, 2 * S * H), jnp.float32),
        ],
        compiler_params=pltpu.CompilerParams(
            vmem_limit_bytes=110 * 1024 * 1024,
        ),
    )(perm, counts, offs, nsteps, X, M,
      bf(W1.T), b1.reshape(1, H), bf(W2.T), b2.reshape(1, I),
      bf(wxc), bf(whc), bc,
      bf(Ws.T[:I]), bf(Ws.T[I:]), bs.reshape(1, H))
    loss = out[0][0]
    ratio = out[1][0]
    return (loss, ratio)


# speedup vs baseline: 1.0209x; 1.0209x over previous
import math

import jax
import jax.numpy as jnp
from jax import lax
from jax.experimental import pallas as pl
from jax.experimental.pallas import tpu as pltpu
from jax.experimental.pallas import tpu_sc as plsc

H = 512
I = 128
S = 7
B = 8
L = 2048

LOG10_HALF = math.log(10.0) / 2.0
LOG_TAU = [s * LOG10_HALF for s in range(S)]
TAU = [math.exp(v) for v in LOG_TAU]


def _pick_row(blk, rem):
    rowmask = lax.broadcasted_iota(jnp.int32, blk.shape, 0) == rem
    return jnp.sum(jnp.where(rowmask, blk, 0.0), axis=0, keepdims=True)


def _routing_sc_kernel(bidx_hbm, perm_hbm, meta_hbm, bidx_v, perm_v, meta_v):
    cid = lax.axis_index("c")
    sid = lax.axis_index("s")

    @pl.when(jnp.logical_and(cid == 0, sid == 0))
    def _():
        pltpu.sync_copy(bidx_hbm, bidx_v)
        iota = lax.broadcasted_iota(jnp.int32, (16,), 0)
        zero = jnp.zeros((16,), jnp.int32)
        zs = jnp.int32(0)

        def count_body(k, cnts):
            v = bidx_v[pl.ds(pl.multiple_of(k * 16, 16), 16)]
            return tuple(cnts[b] + jnp.sum((v == b).astype(jnp.int32))
                         for b in range(B))

        cnts = lax.fori_loop(0, L // 16, count_body, (zs,) * B)
        offs = []
        run = zs
        for b in range(B):
            offs.append(run)
            run = run + cnts[b]

        def scatter_body(k, pos):
            base = k * 16
            v = bidx_v[pl.ds(pl.multiple_of(base, 16), 16)]
            ids = iota + base
            newpos = []
            for b in range(B):
                mask = v == b
                mi = mask.astype(jnp.int32)
                ranks = plsc.cumsum(mi)
                plsc.store_scatter(perm_v, [pos[b] + ranks - 1], ids, mask=mask)
                newpos.append(pos[b] + jnp.sum(mi))
            return tuple(newpos)

        lax.fori_loop(0, L // 16, scatter_body, tuple(offs))

        cnt_lane = zero
        off_lane = zero
        for b in range(B):
            sel = iota == b
            cnt_lane = jnp.where(sel, cnts[b], cnt_lane)
            off_lane = jnp.where(sel, offs[b], off_lane)
        nsteps = cnts[0]
        for b in range(1, B):
            nsteps = jnp.maximum(nsteps, cnts[b])
        meta_v[pl.ds(0, 16)] = cnt_lane
        meta_v[pl.ds(16, 16)] = off_lane
        meta_v[pl.ds(32, 16)] = zero + nsteps
        pltpu.sync_copy(perm_v, perm_hbm)
        pltpu.sync_copy(meta_v, meta_hbm)


def _route_events(batch_idx):
    return pl.kernel(
        _routing_sc_kernel,
        out_type=(
            jax.ShapeDtypeStruct((L,), jnp.int32),
            jax.ShapeDtypeStruct((48,), jnp.int32),
        ),
        mesh=plsc.VectorSubcoreMesh(core_axis_name="c", subcore_axis_name="s"),
        scratch_types=[
            pltpu.VMEM((L,), jnp.int32),
            pltpu.VMEM((L,), jnp.int32),
            pltpu.VMEM((48,), jnp.int32),
        ],
        compiler_params=pltpu.CompilerParams(needs_layout_passes=False),
    )(batch_idx)


def _ctgru_batched_kernel(
    perm_ref,
    counts_ref,
    offs_ref,
    nsteps_ref,
    x_ref,
    m_ref,
    w1t_ref,
    b1_ref,
    w2t_ref,
    b2_ref,
    wxc_ref,
    whc_ref,
    bc_ref,
    ws_x_ref,
    ws_h_ref,
    bs_ref,
    loss_ref,
    ratio_ref,
    state_ref,
    lastt_ref,
    acc_ref,
):
    state_ref[...] = jnp.zeros((8 * B, H), jnp.float32)
    acc_ref[...] = jnp.zeros((B, I), jnp.float32)
    for b in range(B):
        lastt_ref[b] = 0.0

    def step(j, carry):
        xs, ms, acts, ivs = [], [], [], []
        for b in range(B):
            nb = counts_ref[b]
            pos = jnp.maximum(offs_ref[b] + jnp.minimum(j, nb - 1), 0)
            t = perm_ref[pos]
            tbase = (t // 8) * 8
            trem = t - tbase
            xs.append(_pick_row(x_ref[pl.ds(tbase, 8), :], trem))
            ms.append(_pick_row(m_ref[pl.ds(tbase, 8), :], trem))
            active = j < nb
            acts.append(jnp.full((1, 1), active.astype(jnp.float32)))
            ot = t.astype(jnp.float32)
            lt = lastt_ref[b]
            ivs.append(jnp.full((1, 1), ot - lt))
            lastt_ref[b] = jnp.where(active, ot, lt)
        x8 = jnp.concatenate(xs, axis=0)
        m8 = jnp.concatenate(ms, axis=0)
        act = jnp.concatenate(acts, axis=0)
        iv = jnp.concatenate(ivs, axis=0)
        actb = act > 0.5

        h8 = state_ref[S * 8:(S + 1) * 8, :]
        h8b = h8.astype(jnp.bfloat16)
        x8b = x8.astype(jnp.bfloat16)

        rz = (jnp.dot(x8b, wxc_ref[...], preferred_element_type=jnp.float32)
              + jnp.dot(h8b, whc_ref[...], preferred_element_type=jnp.float32)
              + bc_ref[...])

        a = jnp.maximum(
            jnp.dot(h8b, w1t_ref[...], preferred_element_type=jnp.float32)
            + b1_ref[...], 0.0)
        p = jnp.dot(a.astype(jnp.bfloat16), w2t_ref[...], preferred_element_type=jnp.float32) + b2_ref[...]
        acc_ref[...] += jnp.abs(x8 - p) * m8 * act

        q = [-jnp.square(rz[:, s * H:(s + 1) * H] - LOG_TAU[s]) for s in range(S)]
        mx = q[0]
        for s in range(1, S):
            mx = jnp.maximum(mx, q[s])
        e = [jnp.exp(q[s] - mx) for s in range(S)]
        den = e[0]
        for s in range(1, S):
            den = den + e[s]
        hh = [state_ref[s * 8:(s + 1) * 8, :] for s in range(S)]
        rsum = e[0] * hh[0]
        for s in range(1, S):
            rsum += e[s] * hh[s]
        rsum = rsum / den

        h_tilde = jnp.tanh(
            jnp.dot(x8b, ws_x_ref[...], preferred_element_type=jnp.float32)
            + jnp.dot(rsum.astype(jnp.bfloat16), ws_h_ref[...], preferred_element_type=jnp.float32)
            + bs_ref[...])

        Z0 = S * H
        qz = [-jnp.square(rz[:, Z0 + s * H:Z0 + (s + 1) * H] - LOG_TAU[s]) for s in range(S)]
        mz = qz[0]
        for s in range(1, S):
            mz = jnp.maximum(mz, qz[s])
        ez = [jnp.exp(qz[s] - mz) for s in range(S)]
        dz = ez[0]
        for s in range(1, S):
            dz = dz + ez[s]

        new_h = hh[0]
        for s in range(1, S):
            new_h = new_h + hh[s]

        for s in range(S):
            z_s = ez[s] / dz
            expf = jnp.exp(-iv / TAU[s])
            new_hh_s = ((1.0 - z_s) * hh[s] + z_s * h_tilde) * expf
            state_ref[s * 8:(s + 1) * 8, :] = jnp.where(actb, new_hh_s, hh[s])
        state_ref[S * 8:(S + 1) * 8, :] = jnp.where(actb, new_h, h8)
        return carry

    lax.fori_loop(0, nsteps_ref[0], step, 0)

    loss = jnp.sum(acc_ref[...])
    tot_m = jnp.sum(m_ref[...])
    loss_ref[0] = loss
    ratio_ref[0] = loss / tot_m


def kernel(obs_times, event_pt, sample_idx, X, M, batch_idx, device, T,
           W1, b1, W2, b2, Wr, br, Ws, bs, Wst, bst):
    perm, meta = _route_events(batch_idx)
    counts = meta[0:B]
    offs = meta[16:16 + B]
    nsteps = meta[32:33]

    def split_sh(W):
        Wp = W.reshape(H, S, I + H).transpose(2, 1, 0).reshape(I + H, S * H)
        return Wp[:I], Wp[I:]

    bf = lambda w: w.astype(jnp.bfloat16)
    wr_x, wr_h = split_sh(Wr)
    wt_x, wt_h = split_sh(Wst)
    wxc = jnp.concatenate([wr_x, wt_x], axis=1)
    whc = jnp.concatenate([wr_h, wt_h], axis=1)
    br_p = br.reshape(H, S).T.reshape(1, S * H)
    bt_p = bst.reshape(H, S).T.reshape(1, S * H)
    bc = jnp.concatenate([br_p, bt_p], axis=1)
    out = pl.pallas_call(
        _ctgru_batched_kernel,
        out_shape=(
            jax.ShapeDtypeStruct((1,), jnp.float32),
            jax.ShapeDtypeStruct((1,), jnp.float32),
        ),
        in_specs=[
            pl.BlockSpec(memory_space=pltpu.SMEM) for _ in range(4)
        ] + [
            pl.BlockSpec(memory_space=pltpu.VMEM) for _ in range(12)
        ],
        out_specs=(
            pl.BlockSpec(memory_space=pltpu.SMEM),
            pl.BlockSpec(memory_space=pltpu.SMEM),
        ),
        scratch_shapes=[
            pltpu.VMEM((8 * B, H), jnp.float32),
            pltpu.SMEM((B,), jnp.float32),
            pltpu.VMEM((B, I), jnp.float32),
        ],
        compiler_params=pltpu.CompilerParams(
            vmem_limit_bytes=110 * 1024 * 1024,
        ),
    )(perm, counts, offs, nsteps, X, M,
      bf(W1.T), b1.reshape(1, H), bf(W2.T), b2.reshape(1, I),
      bf(wxc), bf(whc), bc,
      bf(Ws.T[:I]), bf(Ws.T[I:]), bs.reshape(1, H))
    loss = out[0][0]
    ratio = out[1][0]
    return (loss, ratio)
